# baseline (speedup 1.0000x reference)
import functools

import jax
import jax.numpy as jnp
from jax import lax
from jax.experimental import pallas as pl
from jax.experimental.pallas import tpu as pltpu
from jax.experimental.pallas import tpu_sc as plsc

BATCH = 4096
EMBED_DIM = 64
TOPIC_SIZE = 128

_info = plsc.get_sparse_core_info()
_NC, _NS = _info.num_cores, _info.num_subcores
_NW = _NC * _NS
_BPW = BATCH // _NW


def _sc_gather_body(users_hbm, items_hbm, utab_hbm, itab_hbm,
                    eu_hbm, ei_hbm,
                    uidx_v, iidx_v, urows_v, irows_v, sem):
    wid = lax.axis_index("s") * _NC + lax.axis_index("c")
    base = wid * _BPW
    pltpu.sync_copy(users_hbm.at[pl.ds(base, _BPW)], uidx_v)
    pltpu.sync_copy(items_hbm.at[pl.ds(base, _BPW)], iidx_v)
    cu = pltpu.async_copy(utab_hbm.at[uidx_v], urows_v, sem)
    ci = pltpu.async_copy(itab_hbm.at[iidx_v], irows_v, sem)
    cu.wait()
    ci.wait()
    pltpu.sync_copy(urows_v, eu_hbm.at[pl.ds(base, _BPW)])
    pltpu.sync_copy(irows_v, ei_hbm.at[pl.ds(base, _BPW)])


_sc_gather = functools.partial(
    pl.kernel,
    mesh=plsc.VectorSubcoreMesh(core_axis_name="c", subcore_axis_name="s"),
    out_type=[
        jax.ShapeDtypeStruct((BATCH, EMBED_DIM), jnp.float32),
        jax.ShapeDtypeStruct((BATCH, EMBED_DIM), jnp.float32),
    ],
    scratch_types=[
        pltpu.VMEM((_BPW,), jnp.int32),
        pltpu.VMEM((_BPW,), jnp.int32),
        pltpu.VMEM((_BPW, EMBED_DIM), jnp.float32),
        pltpu.VMEM((_BPW, EMBED_DIM), jnp.float32),
        pltpu.SemaphoreType.DMA,
    ],
    compiler_params=pltpu.CompilerParams(use_tc_tiling_on_sc=False),
)(_sc_gather_body)


def _tc_combine_body(ctx_ref, topic_ref, eu_ref, ei_ref, out_ref):
    ctx = jnp.dot(ctx_ref[...], topic_ref[...],
                  preferred_element_type=jnp.float32)
    out_ref[...] = jnp.sum(eu_ref[...] * (ctx + ei_ref[...]), axis=1)


def _tc_combine(item_ctx, topic_table, eu, ei):
    nblk = 8
    bs = BATCH // nblk
    return pl.pallas_call(
        _tc_combine_body,
        grid=(nblk,),
        in_specs=[
            pl.BlockSpec((bs, TOPIC_SIZE), lambda i: (i, 0)),
            pl.BlockSpec((TOPIC_SIZE, EMBED_DIM), lambda i: (0, 0)),
            pl.BlockSpec((bs, EMBED_DIM), lambda i: (i, 0)),
            pl.BlockSpec((bs, EMBED_DIM), lambda i: (i, 0)),
        ],
        out_specs=pl.BlockSpec((bs,), lambda i: (i,)),
        out_shape=jax.ShapeDtypeStruct((BATCH,), jnp.float32),
    )(item_ctx, topic_table, eu, ei)


@jax.jit
def kernel(users, items, item_ctx, user_table, item_table, topic_table):
    eu, ei = _sc_gather(users, items, user_table, item_table)
    return _tc_combine(item_ctx, topic_table, eu, ei)


# speedup vs baseline: 1.4278x; 1.4278x over previous
import functools

import jax
import jax.numpy as jnp
from jax import lax
from jax.experimental import pallas as pl
from jax.experimental.pallas import tpu as pltpu
from jax.experimental.pallas import tpu_sc as plsc

BATCH = 4096
EMBED_DIM = 64
TOPIC_SIZE = 128
TABLE_ROWS = 100000
TPR = 8
NTILES = TABLE_ROWS // TPR

_info = plsc.get_sparse_core_info()
_NC, _NS = _info.num_cores, _info.num_subcores
_NW = _NC * _NS
_BPW = BATCH // _NW
_CHUNK = 32
_NCHUNK = _BPW // _CHUNK


def _sc_body(users_hbm, items_hbm, utab_hbm, itab_hbm, eu_hbm, ei_hbm,
             uid_v, iid_v, urows_v, irows_v, sem):
    wid = lax.axis_index("s") * _NC + lax.axis_index("c")
    base = wid * _BPW
    pltpu.sync_copy(users_hbm.at[pl.ds(base, _BPW)], uid_v)
    pltpu.sync_copy(items_hbm.at[pl.ds(base, _BPW)], iid_v)

    copies = []
    for g in range(_BPW // 16):
        uvec = uid_v[pl.ds(g * 16, 16)]
        ivec = iid_v[pl.ds(g * 16, 16)]
        for l in range(16):
            jj = g * 16 + l
            copies.append(pltpu.async_copy(
                utab_hbm.at[uvec[l]], urows_v.at[jj], sem))
            copies.append(pltpu.async_copy(
                itab_hbm.at[ivec[l]], irows_v.at[jj], sem))
    for cp in copies:
        cp.wait()
    pltpu.sync_copy(urows_v, eu_hbm.at[pl.ds(base, _BPW)])
    pltpu.sync_copy(irows_v, ei_hbm.at[pl.ds(base, _BPW)])


_sc_gather = functools.partial(
    pl.kernel,
    mesh=plsc.VectorSubcoreMesh(core_axis_name="c", subcore_axis_name="s"),
    out_type=[
        jax.ShapeDtypeStruct((BATCH, EMBED_DIM), jnp.float32),
        jax.ShapeDtypeStruct((BATCH, EMBED_DIM), jnp.float32),
    ],
    scratch_types=[
        pltpu.VMEM((_BPW,), jnp.int32),
        pltpu.VMEM((_BPW,), jnp.int32),
        pltpu.VMEM((_BPW, EMBED_DIM), jnp.float32),
        pltpu.VMEM((_BPW, EMBED_DIM), jnp.float32),
        pltpu.SemaphoreType.DMA,
    ],
)(_sc_body)


def _tc_combine_body(ctx_ref, topic_ref, eu_ref, ei_ref, out_ref):
    ctx = jnp.dot(ctx_ref[...], topic_ref[...],
                  preferred_element_type=jnp.float32)
    out_ref[...] = jnp.sum(eu_ref[...] * (ctx + ei_ref[...]), axis=1)


def _tc_combine(item_ctx, topic_table, eu, ei):
    nblk = 8
    bs = BATCH // nblk
    return pl.pallas_call(
        _tc_combine_body,
        grid=(nblk,),
        in_specs=[
            pl.BlockSpec((bs, TOPIC_SIZE), lambda i: (i, 0)),
            pl.BlockSpec((TOPIC_SIZE, EMBED_DIM), lambda i: (0, 0)),
            pl.BlockSpec((bs, EMBED_DIM), lambda i: (i, 0)),
            pl.BlockSpec((bs, EMBED_DIM), lambda i: (i, 0)),
        ],
        out_specs=pl.BlockSpec((bs,), lambda i: (i,)),
        out_shape=jax.ShapeDtypeStruct((BATCH,), jnp.float32),
    )(item_ctx, topic_table, eu, ei)


@jax.jit
def kernel(users, items, item_ctx, user_table, item_table, topic_table):
    eu, ei = _sc_gather(users, items, user_table, item_table)
    return _tc_combine(item_ctx, topic_table, eu, ei)
